# baseline (speedup 1.0000x reference)
import numpy as np
import jax
import jax.numpy as jnp
from jax.experimental import pallas as pl
from jax.experimental.pallas import tpu as pltpu

_PATCH = 8
_NBUF = 4


def _extract_body(x_hbm, m_ref, wt_ref, s4_ref, rp4_ref, rp2_ref,
                  s2_ref, oi_ref, om_ref, xbuf, sems):
    n_lat, n_ch = wt_ref.shape[0], wt_ref.shape[1]
    H, W = x_hbm.shape[2], x_hbm.shape[3]
    Wm = s4_ref.shape[1]
    Hm = rp4_ref.shape[0]
    Hl = rp2_ref.shape[0]
    hs = H // _PATCH

    b = pl.program_id(0)
    nb = pl.num_programs(0)

    @pl.when(b == 0)
    def _prologue():
        for k in range(_NBUF):
            pltpu.make_async_copy(x_hbm.at[k], xbuf.at[k], sems.at[k]).start()

    slot = jax.lax.rem(b, _NBUF)
    pltpu.make_async_copy(x_hbm.at[b], xbuf.at[slot], sems.at[slot]).wait()

    xb = [xbuf[slot, c].astype(jnp.bfloat16).reshape(hs, _PATCH, W)
          for c in range(n_ch)]
    mb = m_ref[0].astype(jnp.bfloat16)

    ts = []
    for n in range(n_lat):
        y = xb[0] * wt_ref[n, 0][None, :, :]
        for c in range(1, n_ch):
            y = y + xb[c] * wt_ref[n, c][None, :, :]
        ts.append(jnp.dot(rp4_ref[...], y.reshape(H, W),
                          preferred_element_type=jnp.float32))
    t4 = jnp.concatenate(ts, axis=0).astype(jnp.bfloat16)

    z = jnp.dot(t4, s4_ref[...],
                preferred_element_type=jnp.float32)
    zb = z.astype(jnp.bfloat16)

    pieces = []
    for n in range(n_lat):
        zn = zb[n * Hm:(n + 1) * Hm]
        pieces.append(zn)
        pieces.append(zn * mb)
    tall = jnp.concatenate(pieces, axis=1)
    t2 = jnp.dot(rp2_ref[...], tall,
                 preferred_element_type=jnp.float32)

    s = jnp.concatenate(
        [t2[:, k * Wm:(k + 1) * Wm] for k in range(2 * n_lat)], axis=0)
    o = jnp.dot(s, s2_ref[...],
                preferred_element_type=jnp.float32)

    for n in range(n_lat):
        oi_ref[0, n] = o[2 * n * Hl:(2 * n + 1) * Hl].astype(oi_ref.dtype)
        om_ref[0, n] = o[(2 * n + 1) * Hl:
                         (2 * n + 2) * Hl].astype(om_ref.dtype)

    @pl.when(b + _NBUF < nb)
    def _prefetch():
        pltpu.make_async_copy(x_hbm.at[b + _NBUF], xbuf.at[slot],
                              sems.at[slot]).start()


def kernel(images, ref_masks, w_kernel):
    B, C, H, W = images.shape
    _, Hm, Wm = ref_masks.shape
    N = w_kernel.shape[0]
    Hl, Wl = H // _PATCH, W // _PATCH
    qh, qw = H // Hm, W // Wm
    cph = _PATCH // qh
    cpw = _PATCH // qw
    dt = images.dtype
    bf = jnp.bfloat16

    t8 = np.equal(np.arange(W)[None, :] % _PATCH,
                  np.arange(_PATCH)[:, None]).astype(np.float32)
    wt = jnp.einsum('ncjd,dw->ncjw',
                    w_kernel.reshape(N, C, _PATCH, _PATCH), t8,
                    precision=jax.lax.Precision.HIGHEST).astype(bf)

    s4 = jnp.asarray(np.equal(np.arange(W)[:, None] // qw,
                              np.arange(Wm)[None, :]), dtype=bf)
    rp4 = jnp.asarray(np.equal(np.arange(H)[None, :] // qh,
                               np.arange(Hm)[:, None]), dtype=bf)
    rp2 = jnp.asarray(np.equal(np.arange(Hm)[None, :] // cph,
                               np.arange(Hl)[:, None]), dtype=bf)
    s2 = jnp.asarray(np.equal(np.arange(Wm)[:, None] // cpw,
                              np.arange(Wl)[None, :]),
                     dtype=np.dtype(dt.name) if hasattr(dt, 'name') else dt)

    out_shape = (jax.ShapeDtypeStruct((B, N, Hl, Wl), dt),
                 jax.ShapeDtypeStruct((B, N, Hl, Wl), dt))
    li, lm = pl.pallas_call(
        _extract_body,
        grid=(B,),
        in_specs=[
            pl.BlockSpec(memory_space=pl.ANY),
            pl.BlockSpec((1, Hm, Wm), lambda b: (b, 0, 0)),
            pl.BlockSpec((N, C, _PATCH, W), lambda b: (0, 0, 0, 0)),
            pl.BlockSpec((W, Wm), lambda b: (0, 0)),
            pl.BlockSpec((Hm, H), lambda b: (0, 0)),
            pl.BlockSpec((Hl, Hm), lambda b: (0, 0)),
            pl.BlockSpec((Wm, Wl), lambda b: (0, 0)),
        ],
        out_specs=(pl.BlockSpec((1, N, Hl, Wl), lambda b: (b, 0, 0, 0)),
                   pl.BlockSpec((1, N, Hl, Wl), lambda b: (b, 0, 0, 0))),
        out_shape=out_shape,
        scratch_shapes=[pltpu.VMEM((_NBUF, C, H, W), jnp.float32),
                        pltpu.SemaphoreType.DMA((_NBUF,))],
        compiler_params=pltpu.CompilerParams(
            dimension_semantics=("arbitrary",)),
    )(images, ref_masks, wt, s4, rp4, rp2, s2)
    return li, lm
